# Initial kernel scaffold; baseline (speedup 1.0000x reference)
import jax
import jax.numpy as jnp
from jax.experimental import pallas as pl


def kernel(img_lr, msb_weight):
    raise NotImplementedError("write your pallas kernel here")


# speedup vs baseline: 172.4749x; 172.4749x over previous
import functools
import jax
import jax.numpy as jnp
from jax import lax
from jax.experimental import pallas as pl
from jax.experimental.pallas import tpu as pltpu
from jax.experimental.pallas import tpu_sc as plsc

_L = 16
_H = 512
_HP = _H + 4
_NW = 32
_RW = _H // _NW
_NTBL = 12 * 4096 * 2
_BIAS = 2048
_BIAS12 = 12 * _BIAS


def _rot_off(d, r):
    dx, dy = d
    if r == 0:
        return (dx, dy)
    if r == 1:
        return (dy, -dx)
    if r == 2:
        return (-dx, -dy)
    return (-dy, dx)


_BASE_OFF = {0: ((0, 1), (0, 2)), 1: ((1, 1), (2, 2)), 2: ((1, 2), (2, 1))}
_PERMS = {0: [0, 1, 2, 3], 1: [2, 0, 3, 1], 2: [3, 2, 1, 0], 3: [1, 3, 0, 2]}
_PASS_OFFS = [
    (_rot_off(_BASE_OFF[kt][0], r), _rot_off(_BASE_OFF[kt][1], r))
    for kt in range(3)
    for r in range(4)
]


def _body(p_hbm, t_hbm, out_hbm, img_v, tbl_v, row0_v, row1_v, sem0, sem1):
    wid = lax.axis_index("s") * 2 + lax.axis_index("c")
    pltpu.sync_copy(t_hbm, tbl_v)
    pltpu.sync_copy(p_hbm.at[pl.ds(wid * (_RW * _HP), (_RW + 4) * _HP)], img_v)

    iota = lax.iota(jnp.int32, 16)
    iota2 = iota * 2

    def do_chunk(k, rowbuf):
        def row_body(i, carry):
            li = k * 4 + i

            def vec_body(j0, carry2):
                c0 = j0 * 16
                base2 = (li + 2) * _HP + c0 + 2

                def ld(dx, dy):
                    return img_v[pl.ds(base2 + (dx * _HP + dy), 16)]

                a = ld(0, 0)
                a9 = a << 9
                acc_a = jnp.zeros((16,), jnp.int32)
                acc_b = jnp.zeros((16,), jnp.int32)
                for p, (vb, vc) in enumerate(_PASS_OFFS):
                    bv = ld(vb[0], vb[1])
                    cv = ld(vc[0], vc[1])
                    w0 = a9 + (bv << 5) + (cv << 1) + (p * 8192)
                    acc_a = acc_a + plsc.load_gather(tbl_v, [w0])
                    acc_b = acc_b + plsc.load_gather(tbl_v, [w0 + 1])

                col = iota2 + (2 * c0)
                r0 = jnp.full((16,), 2 * i, jnp.int32)
                r1 = jnp.full((16,), 2 * i + 1, jnp.int32)
                third = jnp.float32(1.0 / 3.0)
                for acc, rr in ((acc_a, r0), (acc_b, r1)):
                    lo = (acc & 0xFFFF) - _BIAS12
                    hi = lax.shift_right_logical(acc, 16) - _BIAS12
                    v0 = lo.astype(jnp.float32) * third
                    v1 = hi.astype(jnp.float32) * third
                    plsc.store_scatter(rowbuf, [rr, col], v0)
                    plsc.store_scatter(rowbuf, [rr, col + 1], v1)
                return carry2

            return lax.fori_loop(0, 32, vec_body, carry)

        lax.fori_loop(0, 4, row_body, 0)

    handles = [None, None]
    bufs = (row0_v, row1_v)
    sems = (sem0, sem1)
    for k in range(4):
        b = k % 2
        if handles[b] is not None:
            handles[b].wait()
        do_chunk(k, bufs[b])
        handles[b] = pltpu.async_copy(
            bufs[b], out_hbm.at[pl.ds(wid * 32 + k * 8, 8)], sems[b]
        )
    handles[0].wait()
    handles[1].wait()


@jax.jit
def kernel(img_lr, msb_weight):
    padded = jnp.pad(img_lr, 2, mode="reflect").astype(jnp.int32).reshape(-1)

    w_int = msb_weight.astype(jnp.int32)
    tabs = []
    for kt in range(3):
        for r in range(4):
            t = w_int[kt][:, jnp.array(_PERMS[r])] + _BIAS
            w0 = t[:, 0] | (t[:, 1] << 16)
            w1 = t[:, 2] | (t[:, 3] << 16)
            tabs.append(jnp.stack([w0, w1], axis=-1).reshape(-1))
    table = jnp.concatenate(tabs)

    mesh = plsc.VectorSubcoreMesh(core_axis_name="c", subcore_axis_name="s")
    run = functools.partial(
        pl.kernel,
        mesh=mesh,
        compiler_params=pltpu.CompilerParams(needs_layout_passes=False),
        out_type=jax.ShapeDtypeStruct((2 * _H, 2 * _H), jnp.float32),
        scratch_types=[
            pltpu.VMEM(((_RW + 4) * _HP,), jnp.int32),
            pltpu.VMEM((_NTBL,), jnp.int32),
            pltpu.VMEM((8, 2 * _H), jnp.float32),
            pltpu.VMEM((8, 2 * _H), jnp.float32),
            pltpu.SemaphoreType.DMA,
            pltpu.SemaphoreType.DMA,
        ],
    )(_body)
    return run(padded, table)


# speedup vs baseline: 241.8648x; 1.4023x over previous
import functools
import jax
import jax.numpy as jnp
from jax import lax
from jax.experimental import pallas as pl
from jax.experimental.pallas import tpu as pltpu
from jax.experimental.pallas import tpu_sc as plsc

_L = 16
_H = 512
_HP = _H + 4
_NW = 32
_RW = _H // _NW
_NTBL = 12 * 4096 * 2
_BIAS = 2048
_BIAS12 = 12 * _BIAS


def _rot_off(d, r):
    dx, dy = d
    if r == 0:
        return (dx, dy)
    if r == 1:
        return (dy, -dx)
    if r == 2:
        return (-dx, -dy)
    return (-dy, dx)


_BASE_OFF = {0: ((0, 1), (0, 2)), 1: ((1, 1), (2, 2)), 2: ((1, 2), (2, 1))}
_PERMS = {0: [0, 1, 2, 3], 1: [2, 0, 3, 1], 2: [3, 2, 1, 0], 3: [1, 3, 0, 2]}
_PASS_OFFS = [
    (_rot_off(_BASE_OFF[kt][0], r), _rot_off(_BASE_OFF[kt][1], r))
    for kt in range(3)
    for r in range(4)
]


def _body(p_hbm, t_hbm, out_hbm, img_v, tbl_v, row0_v, row1_v, sem0, sem1):
    wid = lax.axis_index("s") * 2 + lax.axis_index("c")
    pltpu.sync_copy(t_hbm, tbl_v)
    pltpu.sync_copy(p_hbm.at[pl.ds(wid * (_RW * _HP), (_RW + 4) * _HP)], img_v)

    iota = lax.iota(jnp.int32, 16)
    iota2 = iota * 2

    def do_chunk(k, rowbuf):
        def row_body(i, carry):
            li = k * 4 + i

            def vec_body(j0, carry2):
                c0 = j0 * 16
                base2 = (li + 2) * _HP + c0 + 2

                def ld(dx, dy):
                    return img_v[pl.ds(base2 + (dx * _HP + dy), 16)]

                a = ld(0, 0)
                a8 = a << 8
                acc_a = jnp.zeros((16,), jnp.int32)
                acc_b = jnp.zeros((16,), jnp.int32)
                for p, (vb, vc) in enumerate(_PASS_OFFS):
                    bv = ld(vb[0], vb[1])
                    cv = ld(vc[0], vc[1])
                    w0 = a8 + (bv << 4) + cv + (p * 8192)
                    acc_a = acc_a + plsc.load_gather(tbl_v, [w0])
                    acc_b = acc_b + plsc.load_gather(tbl_v, [w0 + 4096])

                col = iota2 + (2 * c0)
                r0 = jnp.full((16,), 2 * i, jnp.int32)
                r1 = jnp.full((16,), 2 * i + 1, jnp.int32)
                third = jnp.float32(1.0 / 3.0)
                for acc, rr in ((acc_a, r0), (acc_b, r1)):
                    lo = (acc & 0xFFFF) - _BIAS12
                    hi = lax.shift_right_logical(acc, 16) - _BIAS12
                    v0 = lo.astype(jnp.float32) * third
                    v1 = hi.astype(jnp.float32) * third
                    plsc.store_scatter(rowbuf, [rr, col], v0)
                    plsc.store_scatter(rowbuf, [rr, col + 1], v1)
                return carry2

            return lax.fori_loop(0, 32, vec_body, carry)

        lax.fori_loop(0, 4, row_body, 0)

    handles = [None, None]
    bufs = (row0_v, row1_v)
    sems = (sem0, sem1)
    for k in range(4):
        b = k % 2
        if handles[b] is not None:
            handles[b].wait()
        do_chunk(k, bufs[b])
        handles[b] = pltpu.async_copy(
            bufs[b], out_hbm.at[pl.ds(wid * 32 + k * 8, 8)], sems[b]
        )
    handles[0].wait()
    handles[1].wait()


@jax.jit
def kernel(img_lr, msb_weight):
    padded = jnp.pad(img_lr, 2, mode="reflect").astype(jnp.int32).reshape(-1)

    w_int = msb_weight.astype(jnp.int32)
    perm = jnp.array([_PERMS[r] for r in range(4)], jnp.int32)
    t = jnp.take(w_int, perm, axis=2) + _BIAS
    lo = t[..., 0] | (t[..., 1] << 16)
    hi = t[..., 2] | (t[..., 3] << 16)
    table = jnp.stack([lo, hi], axis=-1).transpose(0, 2, 3, 1).reshape(-1)

    mesh = plsc.VectorSubcoreMesh(core_axis_name="c", subcore_axis_name="s")
    run = functools.partial(
        pl.kernel,
        mesh=mesh,
        compiler_params=pltpu.CompilerParams(needs_layout_passes=False),
        out_type=jax.ShapeDtypeStruct((2 * _H, 2 * _H), jnp.float32),
        scratch_types=[
            pltpu.VMEM(((_RW + 4) * _HP,), jnp.int32),
            pltpu.VMEM((_NTBL,), jnp.int32),
            pltpu.VMEM((8, 2 * _H), jnp.float32),
            pltpu.VMEM((8, 2 * _H), jnp.float32),
            pltpu.SemaphoreType.DMA,
            pltpu.SemaphoreType.DMA,
        ],
    )(_body)
    return run(padded, table)
